# baseline (speedup 1.0000x reference)
import functools

import jax
import jax.numpy as jnp
from jax import lax
from jax.experimental import pallas as pl
from jax.experimental.pallas import tpu as pltpu
from jax.experimental.pallas import tpu_sc as plsc

NUM_ROWS = 1_000_000
EMBED = 2
B = 16384 * 200
LANE = 128
NW = 32
ROWS128 = B // LANE
ROWS_PER_W = ROWS128 // NW
K = 80
NCHUNK = ROWS_PER_W // K

S1_ROWS = 2048
S1_WORDS = S1_ROWS * EMBED
S1_NCHUNKS = -(-NUM_ROWS // S1_ROWS)
S1_PER_W = -(-S1_NCHUNKS // NW)

_MESH = plsc.VectorSubcoreMesh(core_axis_name="c", subcore_axis_name="s")


_PAD = 16


@functools.partial(
    pl.kernel,
    out_type=jax.ShapeDtypeStruct((NUM_ROWS * EMBED,), jnp.float32),
    mesh=_MESH,
    scratch_types=[
        pltpu.VMEM((_PAD + S1_WORDS + _PAD,), jnp.float32),
        pltpu.VMEM((S1_WORDS,), jnp.float32),
    ],
)
def _softmax_table(tflat, qflat, buf, obuf):
    wid = lax.axis_index("s") * 2 + lax.axis_index("c")
    even = (lax.iota(jnp.int32, 16) & 1) == 0

    def chunk_body(t, carry):
        cid = jnp.minimum(wid + NW * t, S1_NCHUNKS - 1)
        start = jnp.minimum(cid * S1_ROWS, NUM_ROWS - S1_ROWS) * EMBED
        pltpu.sync_copy(tflat.at[pl.ds(start, S1_WORDS)],
                        buf.at[pl.ds(_PAD, S1_WORDS)])

        def grp(g, c2):
            base = _PAD + g * 16
            v = buf[pl.ds(base, 16)]
            nxt = buf[pl.ds(base + 1, 16)]
            prv = buf[pl.ds(base - 1, 16)]
            sw = jnp.where(even, nxt, prv)
            obuf[pl.ds(g * 16, 16)] = 1.0 / (1.0 + jnp.exp(sw - v))
            return c2

        lax.fori_loop(0, S1_WORDS // 16, grp, 0)
        pltpu.sync_copy(obuf, qflat.at[pl.ds(start, S1_WORDS)])
        return carry

    lax.fori_loop(0, S1_PER_W, chunk_body, 0)


@functools.partial(
    pl.kernel,
    out_type=jax.ShapeDtypeStruct((B, EMBED), jnp.float32),
    mesh=_MESH,
    scratch_types=[
        pltpu.VMEM((K, LANE), jnp.int32),
        pltpu.VMEM((K * LANE, EMBED), jnp.float32),
        pltpu.SemaphoreType.DMA,
    ],
    compiler_params=pltpu.CompilerParams(use_tc_tiling_on_sc=False),
)
def _gather_rows(xr, q, out, idx_v, rows_v, sem):
    wid = lax.axis_index("s") * 2 + lax.axis_index("c")

    def chunk(c, carry):
        base = wid * ROWS_PER_W + c * K
        pltpu.sync_copy(xr.at[pl.ds(base, K)], idx_v)

        def fire(j, c2):
            pltpu.make_async_copy(q.at[idx_v.at[j]],
                                  rows_v.at[pl.ds(j * LANE, LANE)], sem).start()
            return c2

        lax.fori_loop(0, K, fire, 0)

        def drain(j, c2):
            pltpu.make_async_copy(q.at[idx_v.at[j]],
                                  rows_v.at[pl.ds(j * LANE, LANE)], sem).wait()
            return c2

        lax.fori_loop(0, K, drain, 0)
        pltpu.sync_copy(rows_v, out.at[pl.ds(base * LANE, K * LANE)])
        return carry

    lax.fori_loop(0, NCHUNK, chunk, 0)


def kernel(x, table):
    qflat = _softmax_table(table.reshape(-1))
    q = qflat.reshape(NUM_ROWS, EMBED)
    xr = x.reshape(ROWS128, LANE)
    return _gather_rows(xr, q)


# speedup vs baseline: 12.5460x; 12.5460x over previous
import functools

import jax
import jax.numpy as jnp
from jax import lax
from jax.experimental import pallas as pl
from jax.experimental.pallas import tpu as pltpu
from jax.experimental.pallas import tpu_sc as plsc

NUM_ROWS = 1_000_000
B = 16384 * 200
LANE = 128
NW = 32
ROWS128 = B // LANE
ROWS_PER_W = ROWS128 // NW
K = 80
NCHUNK = ROWS_PER_W // K

S1_CH = 2048
S1_NCHUNKS = -(-NUM_ROWS // S1_CH)
S1_PER_W = -(-S1_NCHUNKS // NW)

_MESH = plsc.VectorSubcoreMesh(core_axis_name="c", subcore_axis_name="s")


@functools.partial(
    pl.kernel,
    out_type=(
        jax.ShapeDtypeStruct((NUM_ROWS,), jnp.float32),
        jax.ShapeDtypeStruct((NUM_ROWS,), jnp.float32),
    ),
    mesh=_MESH,
    scratch_types=[
        pltpu.VMEM((S1_CH,), jnp.float32),
        pltpu.VMEM((S1_CH,), jnp.float32),
        pltpu.VMEM((S1_CH,), jnp.float32),
        pltpu.VMEM((S1_CH,), jnp.float32),
    ],
    compiler_params=pltpu.CompilerParams(use_tc_tiling_on_sc=False),
)
def _softmax_table(tt, qa, qb, abuf, bbuf, pabuf, pbbuf):
    wid = lax.axis_index("s") * 2 + lax.axis_index("c")

    def chunk_body(t, carry):
        cid = jnp.minimum(wid + NW * t, S1_NCHUNKS - 1)
        start = jnp.minimum(cid * S1_CH, NUM_ROWS - S1_CH)
        pltpu.sync_copy(tt.at[0, pl.ds(start, S1_CH)], abuf)
        pltpu.sync_copy(tt.at[1, pl.ds(start, S1_CH)], bbuf)

        def grp(g, c2):
            a = abuf[pl.ds(g * 16, 16)]
            b = bbuf[pl.ds(g * 16, 16)]
            pabuf[pl.ds(g * 16, 16)] = 1.0 / (1.0 + jnp.exp(b - a))
            pbbuf[pl.ds(g * 16, 16)] = 1.0 / (1.0 + jnp.exp(a - b))
            return c2

        lax.fori_loop(0, S1_CH // 16, grp, 0)
        pltpu.sync_copy(pabuf, qa.at[pl.ds(start, S1_CH)])
        pltpu.sync_copy(pbbuf, qb.at[pl.ds(start, S1_CH)])
        return carry

    lax.fori_loop(0, S1_PER_W, chunk_body, 0)


@functools.partial(
    pl.kernel,
    out_type=jax.ShapeDtypeStruct((ROWS128, 2, LANE), jnp.float32),
    mesh=_MESH,
    scratch_types=[
        pltpu.VMEM((K, LANE), jnp.int32),
        pltpu.VMEM((K, LANE), jnp.float32),
        pltpu.VMEM((K, LANE), jnp.float32),
        pltpu.SemaphoreType.DMA,
    ],
    compiler_params=pltpu.CompilerParams(use_tc_tiling_on_sc=False),
)
def _gather_rows(xr, qa, qb, out, idx_v, va, vb, sem):
    wid = lax.axis_index("s") * 2 + lax.axis_index("c")

    def chunk(c, carry):
        base = wid * ROWS_PER_W + c * K
        pltpu.sync_copy(xr.at[pl.ds(base, K)], idx_v)

        def fire(j, c2):
            pltpu.make_async_copy(qa.at[idx_v.at[j]], va.at[j], sem).start()
            pltpu.make_async_copy(qb.at[idx_v.at[j]], vb.at[j], sem).start()
            return c2

        lax.fori_loop(0, K, fire, 0)

        def drain(j, c2):
            pltpu.make_async_copy(qa.at[idx_v.at[j]], va.at[j], sem).wait()
            pltpu.make_async_copy(qb.at[idx_v.at[j]], vb.at[j], sem).wait()
            return c2

        lax.fori_loop(0, K, drain, 0)
        pltpu.sync_copy(va, out.at[pl.ds(base, K), 0])
        pltpu.sync_copy(vb, out.at[pl.ds(base, K), 1])
        return carry

    lax.fori_loop(0, NCHUNK, chunk, 0)


def kernel(x, table):
    tt = table.T
    qa, qb = _softmax_table(tt)
    xr = x.reshape(ROWS128, LANE)
    ob = _gather_rows(xr, qa, qb)
    return ob.transpose(0, 2, 1).reshape(B, 2)


# speedup vs baseline: 17.5619x; 1.3998x over previous
import functools

import jax
import jax.numpy as jnp
from jax import lax
from jax.experimental import pallas as pl
from jax.experimental.pallas import tpu as pltpu
from jax.experimental.pallas import tpu_sc as plsc

NUM_ROWS = 1_000_000
B = 16384 * 200
LANE = 128
NW = 32
ROWS128 = B // LANE
ROWS_PER_W = ROWS128 // NW
K = 80
NCHUNK = ROWS_PER_W // K

S1_CH = 2048
S1_NCHUNKS = -(-NUM_ROWS // S1_CH)
S1_PER_W = -(-S1_NCHUNKS // NW)

_MESH = plsc.VectorSubcoreMesh(core_axis_name="c", subcore_axis_name="s")


@functools.partial(
    pl.kernel,
    out_type=jax.ShapeDtypeStruct((NUM_ROWS,), jnp.float32),
    mesh=_MESH,
    scratch_types=[
        pltpu.VMEM((S1_CH,), jnp.float32),
        pltpu.VMEM((S1_CH,), jnp.float32),
        pltpu.VMEM((S1_CH,), jnp.float32),
    ],
    compiler_params=pltpu.CompilerParams(use_tc_tiling_on_sc=False),
)
def _softmax_table(tt, qs, abuf, bbuf, pbuf):
    wid = lax.axis_index("s") * 2 + lax.axis_index("c")

    def chunk_body(t, carry):
        cid = jnp.minimum(wid + NW * t, S1_NCHUNKS - 1)
        start = jnp.minimum(cid * S1_CH, NUM_ROWS - S1_CH)
        pltpu.sync_copy(tt.at[0, pl.ds(start, S1_CH)], abuf)
        pltpu.sync_copy(tt.at[1, pl.ds(start, S1_CH)], bbuf)

        def grp(g, c2):
            a = abuf[pl.ds(g * 16, 16)]
            b = bbuf[pl.ds(g * 16, 16)]
            pbuf[pl.ds(g * 16, 16)] = 1.0 / (1.0 + jnp.exp(b - a))
            return c2

        lax.fori_loop(0, S1_CH // 16, grp, 0)
        pltpu.sync_copy(pbuf, qs.at[pl.ds(start, S1_CH)])
        return carry

    lax.fori_loop(0, S1_PER_W, chunk_body, 0)


@functools.partial(
    pl.kernel,
    out_type=jax.ShapeDtypeStruct((ROWS128, 2, LANE), jnp.float32),
    mesh=_MESH,
    scratch_types=[
        pltpu.VMEM((K, LANE), jnp.int32),
        pltpu.VMEM((K, LANE), jnp.float32),
        pltpu.VMEM((K, LANE), jnp.float32),
        pltpu.SemaphoreType.DMA,
    ],
    compiler_params=pltpu.CompilerParams(use_tc_tiling_on_sc=False),
)
def _gather_rows(xr, qs, out, idx_v, va, vb, sem):
    wid = lax.axis_index("s") * 2 + lax.axis_index("c")

    def chunk(c, carry):
        base = wid * ROWS_PER_W + c * K
        pltpu.sync_copy(xr.at[pl.ds(base, K)], idx_v)

        def fire(j, c2):
            pltpu.make_async_copy(qs.at[idx_v.at[j]], va.at[j], sem).start()
            return c2

        lax.fori_loop(0, K, fire, 0)

        def drain(j, c2):
            pltpu.make_async_copy(qs.at[idx_v.at[j]], va.at[j], sem).wait()
            return c2

        lax.fori_loop(0, K, drain, 0)

        def flip(t, c2):
            j = t >> 3
            col = (t & 7) * 16
            vb[j, pl.ds(col, 16)] = 1.0 - va[j, pl.ds(col, 16)]
            return c2

        lax.fori_loop(0, K * (LANE // 16), flip, 0)
        pltpu.sync_copy(va, out.at[pl.ds(base, K), 0])
        pltpu.sync_copy(vb, out.at[pl.ds(base, K), 1])
        return carry

    lax.fori_loop(0, NCHUNK, chunk, 0)


def kernel(x, table):
    tt = table.T
    qs = _softmax_table(tt)
    xr = x.reshape(ROWS128, LANE)
    ob = _gather_rows(xr, qs)
    return ob.transpose(0, 2, 1).reshape(B, 2)


# speedup vs baseline: 21.2635x; 1.2108x over previous
import functools

import jax
import jax.numpy as jnp
from jax import lax
from jax.experimental import pallas as pl
from jax.experimental.pallas import tpu as pltpu
from jax.experimental.pallas import tpu_sc as plsc

NUM_ROWS = 1_000_000
B = 16384 * 200
LANE = 128
NW = 32
ROWS128 = B // LANE
ROWS_PER_W = ROWS128 // NW
K = 160
NCHUNK = ROWS_PER_W // K

S1_CH = 8192
S1_NCHUNKS = -(-NUM_ROWS // S1_CH)
S1_PER_W = -(-S1_NCHUNKS // NW)

_MESH = plsc.VectorSubcoreMesh(core_axis_name="c", subcore_axis_name="s")


@functools.partial(
    pl.kernel,
    out_type=jax.ShapeDtypeStruct((NUM_ROWS,), jnp.float32),
    mesh=_MESH,
    scratch_types=[
        pltpu.VMEM((2, S1_CH), jnp.float32),
        pltpu.VMEM((2, S1_CH), jnp.float32),
        pltpu.VMEM((2, S1_CH), jnp.float32),
        pltpu.SemaphoreType.DMA,
        pltpu.SemaphoreType.DMA,
        pltpu.SemaphoreType.DMA,
    ],
    compiler_params=pltpu.CompilerParams(use_tc_tiling_on_sc=False),
)
def _softmax_table(tt, qs, abuf, bbuf, pbuf, semi, semo0, semo1):
    wid = lax.axis_index("s") * 2 + lax.axis_index("c")
    semo = (semo0, semo1)

    def chunk_start(t):
        cid = jnp.minimum(wid + NW * t, S1_NCHUNKS - 1)
        return jnp.minimum(cid * S1_CH, NUM_ROWS - S1_CH)

    def load_start(t, s):
        start = chunk_start(t)
        pltpu.make_async_copy(tt.at[0, pl.ds(start, S1_CH)], abuf.at[s], semi).start()
        pltpu.make_async_copy(tt.at[1, pl.ds(start, S1_CH)], bbuf.at[s], semi).start()

    def load_wait(t, s):
        start = chunk_start(t)
        pltpu.make_async_copy(tt.at[0, pl.ds(start, S1_CH)], abuf.at[s], semi).wait()
        pltpu.make_async_copy(tt.at[1, pl.ds(start, S1_CH)], bbuf.at[s], semi).wait()

    def compute(s):
        def grp(g, c2):
            a = abuf[s, pl.ds(g * 16, 16)]
            b = bbuf[s, pl.ds(g * 16, 16)]
            pbuf[s, pl.ds(g * 16, 16)] = 1.0 / (1.0 + jnp.exp(b - a))
            return c2
        lax.fori_loop(0, S1_CH // 16, grp, 0)

    def store_start(t, s):
        pltpu.make_async_copy(pbuf.at[s], qs.at[pl.ds(chunk_start(t), S1_CH)],
                              semo[s]).start()

    def store_wait(t, s):
        pltpu.make_async_copy(pbuf.at[s], qs.at[pl.ds(chunk_start(t), S1_CH)],
                              semo[s]).wait()

    load_start(0, 0)
    for t in range(S1_PER_W):
        s = t & 1
        if t + 1 < S1_PER_W:
            load_start(t + 1, 1 - s)
        load_wait(t, s)
        if t >= 2:
            store_wait(t - 2, s)
        compute(s)
        store_start(t, s)
    store_wait(S1_PER_W - 2, S1_PER_W & 1)
    store_wait(S1_PER_W - 1, (S1_PER_W - 1) & 1)


@functools.partial(
    pl.kernel,
    out_type=jax.ShapeDtypeStruct((ROWS128, 2, LANE), jnp.float32),
    mesh=_MESH,
    scratch_types=[
        pltpu.VMEM((2, K, LANE), jnp.int32),
        pltpu.VMEM((K, LANE), jnp.float32),
        pltpu.VMEM((K, LANE), jnp.float32),
        pltpu.VMEM((K, LANE), jnp.float32),
        pltpu.VMEM((K, LANE), jnp.float32),
        pltpu.SemaphoreType.DMA,
        pltpu.SemaphoreType.DMA,
        pltpu.SemaphoreType.DMA,
    ],
)
def _gather_rows(xr, qs, out, idx_v, va0, vb0, va1, vb1, semg, semo0, semo1):
    wid = lax.axis_index("s") * 2 + lax.axis_index("c")
    w0 = wid * ROWS_PER_W
    va = (va0, va1)
    vb = (vb0, vb1)
    semo = (semo0, semo1)

    def fire(c, s):
        def body(j, c2):
            pltpu.make_async_copy(qs.at[idx_v.at[s, j]], va[s].at[j], semg).start()
            return c2
        lax.fori_loop(0, K, body, 0)

    def drain(c, s):
        def body(j, c2):
            pltpu.make_async_copy(qs.at[idx_v.at[s, j]], va[s].at[j], semg).wait()
            return c2
        lax.fori_loop(0, K, body, 0)

    def flip(s):
        def body(t, c2):
            j = t >> 3
            col = (t & 7) * 16
            vb[s][j, pl.ds(col, 16)] = 1.0 - va[s][j, pl.ds(col, 16)]
            return c2
        lax.fori_loop(0, K * (LANE // 16), body, 0)

    def out_start(c, s):
        base = w0 + c * K
        pltpu.make_async_copy(va[s], out.at[pl.ds(base, K), 0], semo[s]).start()
        pltpu.make_async_copy(vb[s], out.at[pl.ds(base, K), 1], semo[s]).start()

    def out_wait(c, s):
        base = w0 + c * K
        pltpu.make_async_copy(va[s], out.at[pl.ds(base, K), 0], semo[s]).wait()
        pltpu.make_async_copy(vb[s], out.at[pl.ds(base, K), 1], semo[s]).wait()

    pltpu.sync_copy(xr.at[pl.ds(w0, K)], idx_v.at[0])
    fire(0, 0)
    for c in range(NCHUNK):
        s = c & 1
        drain(c, s)
        if c + 1 < NCHUNK:
            pltpu.sync_copy(xr.at[pl.ds(w0 + (c + 1) * K, K)], idx_v.at[1 - s])
            if c >= 1:
                out_wait(c - 1, 1 - s)
            fire(c + 1, 1 - s)
        flip(s)
        out_start(c, s)
    out_wait(NCHUNK - 2, NCHUNK & 1)
    out_wait(NCHUNK - 1, (NCHUNK - 1) & 1)


def kernel(x, table):
    tt = table.T
    qs = _softmax_table(tt)
    xr = x.reshape(ROWS128, LANE)
    ob = _gather_rows(xr, qs)
    return ob.transpose(0, 2, 1).reshape(B, 2)
